# Initial kernel scaffold; baseline (speedup 1.0000x reference)
import jax
import jax.numpy as jnp
from jax.experimental import pallas as pl


def kernel(x, edge_index, batch, W1, b1, W2, b2, lin_w, lin_b):
    raise NotImplementedError("write your pallas kernel here")


# speedup vs baseline: 18.8802x; 18.8802x over previous
import functools

import jax
import jax.numpy as jnp
from jax import lax
from jax.experimental import pallas as pl
from jax.experimental.pallas import tpu as pltpu
from jax.experimental.pallas import tpu_sc as plsc

N = 10000
E = 320000
D = 128
G = 64
NC = 2
NS = 16
NW = NC * NS
NP = 10240
C = 80
EPW = E // NW
NCH = EPW // C
RPT = NP // NS
BR = 1000
NG = N // BR


def _mesh():
    return plsc.VectorSubcoreMesh(core_axis_name="c", subcore_axis_name="s")


def _deg(dst_r):
    @functools.partial(
        pl.kernel,
        out_type=jax.ShapeDtypeStruct((NP,), jnp.float32),
        mesh=_mesh(),
        scratch_types=[
            pltpu.VMEM((NCH, C), jnp.int32),
            pltpu.VMEM((C,), jnp.float32),
            pltpu.VMEM((RPT,), jnp.float32),
            pltpu.VMEM_SHARED((NP,), jnp.float32),
        ],
    )
    def deg_kernel(dst_hbm, out_hbm, idx_v, ones_v, buf_v, acc_sh):
        cid = lax.axis_index("c")
        sid = lax.axis_index("s")

        def fill_ones(i, _):
            ones_v[pl.ds(i * 16, 16)] = jnp.ones((16,), jnp.float32)
            return 0

        lax.fori_loop(0, C // 16, fill_ones, 0)

        def fill_zero(i, _):
            buf_v[pl.ds(i * 16, 16)] = jnp.zeros((16,), jnp.float32)
            return 0

        lax.fori_loop(0, RPT // 16, fill_zero, 0)
        pltpu.sync_copy(buf_v, acc_sh.at[pl.ds(sid * RPT, RPT)])
        plsc.subcore_barrier()

        for r in range(2):
            pltpu.sync_copy(dst_hbm.at[sid * 2 + r], idx_v)

            def chunk(ch, _):
                pltpu.sync_copy(ones_v, acc_sh.at[idx_v.at[ch]], add=True)
                return 0

            lax.fori_loop(0, NCH, chunk, 0)
        plsc.subcore_barrier()

        @pl.when(cid == 0)
        def _():
            pltpu.sync_copy(acc_sh.at[pl.ds(sid * RPT, RPT)], buf_v)
            pltpu.sync_copy(buf_v, out_hbm.at[pl.ds(sid * RPT, RPT)])

    return deg_kernel(dst_r)


def _edge_pass(y, src_r, dst_r):
    @functools.partial(
        pl.kernel,
        out_type=jax.ShapeDtypeStruct((NC * NP, D), jnp.float32),
        mesh=_mesh(),
        scratch_types=[
            pltpu.VMEM((NCH, C), jnp.int32),
            pltpu.VMEM((NCH, C), jnp.int32),
            pltpu.VMEM((C, D), jnp.float32),
            pltpu.SemaphoreType.DMA,
            pltpu.VMEM_SHARED((NP, D), jnp.float32),
        ],
    )
    def edge_kernel(y_hbm, src_hbm, dst_hbm, out_hbm, si_v, di_v, rows_v, sem, acc_sh):
        cid = lax.axis_index("c")
        sid = lax.axis_index("s")
        wid = sid * NC + cid

        def fz(i, _):
            rows_v[i // (D // 16), pl.ds((i % (D // 16)) * 16, 16)] = jnp.zeros(
                (16,), jnp.float32
            )
            return 0

        lax.fori_loop(0, C * (D // 16), fz, 0)
        for r in range(RPT // C):
            pltpu.sync_copy(rows_v, acc_sh.at[pl.ds(sid * RPT + r * C, C)])
        plsc.subcore_barrier()

        pltpu.sync_copy(src_hbm.at[wid], si_v)
        pltpu.sync_copy(dst_hbm.at[wid], di_v)

        def chunk(ch, _):
            pltpu.async_copy(y_hbm.at[si_v.at[ch]], rows_v, sem).wait()
            pltpu.sync_copy(rows_v, acc_sh.at[di_v.at[ch]], add=True)
            return 0

        lax.fori_loop(0, NCH, chunk, 0)
        plsc.subcore_barrier()

        def wb(r, _):
            pltpu.sync_copy(acc_sh.at[pl.ds(sid * RPT + r * C, C)], rows_v)
            pltpu.sync_copy(
                rows_v, out_hbm.at[pl.ds(cid * NP + sid * RPT + r * C, C)]
            )
            return 0

        lax.fori_loop(0, RPT // C, wb, 0)

    return edge_kernel(y, src_r, dst_r)


def _tc_pre(deg_col, x, W1):
    def body(deg_ref, x_ref, w_ref, y_ref, dis_ref):
        dis = lax.rsqrt(deg_ref[...] + 1.0)
        xw = jnp.dot(x_ref[...], w_ref[...], preferred_element_type=jnp.float32)
        y_ref[...] = xw * dis
        dis_ref[...] = dis

    return pl.pallas_call(
        body,
        grid=(NG,),
        in_specs=[
            pl.BlockSpec((BR, 1), lambda i: (i, 0)),
            pl.BlockSpec((BR, D), lambda i: (i, 0)),
            pl.BlockSpec((D, D), lambda i: (0, 0)),
        ],
        out_specs=[
            pl.BlockSpec((BR, D), lambda i: (i, 0)),
            pl.BlockSpec((BR, 1), lambda i: (i, 0)),
        ],
        out_shape=[
            jax.ShapeDtypeStruct((N, D), jnp.float32),
            jax.ShapeDtypeStruct((N, 1), jnp.float32),
        ],
    )(deg_col, x, W1)


def _tc_mid(acc0, acc1, y1, dis, b1, W2):
    def body(a0, a1, y, d, b, w, y2_ref):
        h = jnp.maximum((a0[...] + a1[...] + y[...]) * d[...] + b[...], 0.0)
        y2_ref[...] = (
            jnp.dot(h, w[...], preferred_element_type=jnp.float32) * d[...]
        )

    return pl.pallas_call(
        body,
        grid=(NG,),
        in_specs=[
            pl.BlockSpec((BR, D), lambda i: (i, 0)),
            pl.BlockSpec((BR, D), lambda i: (i, 0)),
            pl.BlockSpec((BR, D), lambda i: (i, 0)),
            pl.BlockSpec((BR, 1), lambda i: (i, 0)),
            pl.BlockSpec((D,), lambda i: (0,)),
            pl.BlockSpec((D, D), lambda i: (0, 0)),
        ],
        out_specs=pl.BlockSpec((BR, D), lambda i: (i, 0)),
        out_shape=jax.ShapeDtypeStruct((N, D), jnp.float32),
    )(acc0, acc1, y1, dis, b1, W2)


def _tc_post(acc0, acc1, y2, dis, b2, batch_col, lin_wT, lin_b2):
    def body(a0, a1, y, d, b, bat, lwT, lb, sums_ref, counts_ref, out_ref):
        i = pl.program_id(0)
        h = jnp.maximum((a0[...] + a1[...] + y[...]) * d[...] + b[...], 0.0)
        g_iota = lax.broadcasted_iota(jnp.int32, (1, G), 1)
        oh = (bat[...] == g_iota).astype(jnp.float32)
        psum = lax.dot_general(
            oh, h, (((0,), (0,)), ((), ())),
            preferred_element_type=jnp.float32,
        )
        pcnt = jnp.sum(oh, axis=0)[:, None]

        @pl.when(i == 0)
        def _():
            sums_ref[...] = psum
            counts_ref[...] = pcnt

        @pl.when(i > 0)
        def _():
            sums_ref[...] += psum
            counts_ref[...] += pcnt

        @pl.when(i == NG - 1)
        def _():
            pooled = sums_ref[...] / jnp.maximum(counts_ref[...], 1.0)
            out_ref[...] = (
                jnp.sum(pooled * lwT[...], axis=1, keepdims=True) + lb[...]
            )

    _, _, out = pl.pallas_call(
        body,
        grid=(NG,),
        in_specs=[
            pl.BlockSpec((BR, D), lambda i: (i, 0)),
            pl.BlockSpec((BR, D), lambda i: (i, 0)),
            pl.BlockSpec((BR, D), lambda i: (i, 0)),
            pl.BlockSpec((BR, 1), lambda i: (i, 0)),
            pl.BlockSpec((D,), lambda i: (0,)),
            pl.BlockSpec((BR, 1), lambda i: (i, 0)),
            pl.BlockSpec((1, D), lambda i: (0, 0)),
            pl.BlockSpec((1, 1), lambda i: (0, 0)),
        ],
        out_specs=[
            pl.BlockSpec((G, D), lambda i: (0, 0)),
            pl.BlockSpec((G, 1), lambda i: (0, 0)),
            pl.BlockSpec((G, 1), lambda i: (0, 0)),
        ],
        out_shape=[
            jax.ShapeDtypeStruct((G, D), jnp.float32),
            jax.ShapeDtypeStruct((G, 1), jnp.float32),
            jax.ShapeDtypeStruct((G, 1), jnp.float32),
        ],
    )(acc0, acc1, y2, dis, b2, batch_col, lin_wT, lin_b2)
    return out


def kernel(x, edge_index, batch, W1, b1, W2, b2, lin_w, lin_b):
    src_r = edge_index[0].reshape(NW, NCH, C)
    dst_r = edge_index[1].reshape(NW, NCH, C)

    deg = _deg(dst_r)
    deg_col = deg[:N, None]

    y1, dis = _tc_pre(deg_col, x, W1)

    accs1 = _edge_pass(y1, src_r, dst_r)
    y2 = _tc_mid(accs1[:N], accs1[NP:NP + N], y1, dis, b1, W2)

    accs2 = _edge_pass(y2, src_r, dst_r)
    return _tc_post(
        accs2[:N], accs2[NP:NP + N], y2, dis, b2,
        batch[:, None], lin_w.T, lin_b[:, None],
    )


# speedup vs baseline: 20.7304x; 1.0980x over previous
import functools

import jax
import jax.numpy as jnp
from jax import lax
from jax.experimental import pallas as pl
from jax.experimental.pallas import tpu as pltpu
from jax.experimental.pallas import tpu_sc as plsc

N = 10000
E = 320000
D = 128
G = 64
NC = 2
NS = 16
HN = 5120
TR = 128
AR = HN + TR
C = 80
EPT = E // NS
NCH = EPT // C
RPT = AR // NS
NBUF = 5
NPD = NS * 640
DRT = NPD // NS
BR = 1000
NG = N // BR

_MESH = dict(mesh=plsc.VectorSubcoreMesh(core_axis_name="c", subcore_axis_name="s"))


@functools.partial(
    pl.kernel,
    out_type=jax.ShapeDtypeStruct((NPD,), jnp.float32),
    scratch_types=[
        pltpu.VMEM((NCH, C), jnp.int32),
        pltpu.VMEM((C,), jnp.float32),
        pltpu.VMEM((DRT,), jnp.float32),
        pltpu.VMEM_SHARED((NPD,), jnp.float32),
    ],
    **_MESH,
)
def _deg_kernel(dst_hbm, out_hbm, idx_v, ones_v, buf_v, acc_sh):
    cid = lax.axis_index("c")
    sid = lax.axis_index("s")

    def fill_ones(i, _):
        ones_v[pl.ds(i * 16, 16)] = jnp.ones((16,), jnp.float32)
        return 0

    lax.fori_loop(0, C // 16, fill_ones, 0)

    def fill_zero(i, _):
        buf_v[pl.ds(i * 16, 16)] = jnp.zeros((16,), jnp.float32)
        return 0

    lax.fori_loop(0, DRT // 16, fill_zero, 0)
    pltpu.sync_copy(buf_v, acc_sh.at[pl.ds(sid * DRT, DRT)])
    plsc.subcore_barrier()

    pltpu.sync_copy(dst_hbm.at[sid], idx_v)

    def chunk(ch, _):
        pltpu.sync_copy(ones_v, acc_sh.at[idx_v.at[ch]], add=True)
        return 0

    lax.fori_loop(0, NCH, chunk, 0)
    plsc.subcore_barrier()

    @pl.when(cid == 0)
    def _():
        pltpu.sync_copy(acc_sh.at[pl.ds(sid * DRT, DRT)], buf_v)
        pltpu.sync_copy(buf_v, out_hbm.at[pl.ds(sid * DRT, DRT)])


NST = 2
CHS = NCH // NST


@functools.partial(
    pl.kernel,
    out_type=jax.ShapeDtypeStruct((NC * AR, D), jnp.float32),
    scratch_types=[
        pltpu.VMEM((CHS, C), jnp.int32),
        pltpu.VMEM((CHS, C), jnp.int32),
        pltpu.VMEM((NBUF, C, D), jnp.float32),
        pltpu.SemaphoreType.DMA((NBUF,)),
        pltpu.VMEM_SHARED((AR, D), jnp.float32),
    ],
    **_MESH,
)
def _edge_kernel(y_hbm, z_hbm, src_hbm, dst_hbm, out_hbm, si_v, di_v, rows_v, sems, acc_sh):
    cid = lax.axis_index("c")
    sid = lax.axis_index("s")

    pltpu.sync_copy(z_hbm, acc_sh.at[pl.ds(sid * RPT, RPT)])
    plsc.subcore_barrier()

    for st in range(NST):
        pltpu.sync_copy(src_hbm.at[sid * NST + st], si_v)
        pltpu.sync_copy(dst_hbm.at[(cid * NS + sid) * NST + st], di_v)

        for b in range(NBUF):
            pltpu.async_copy(y_hbm.at[si_v.at[b]], rows_v.at[b], sems.at[b])

        def outer(o, _):
            for b in range(NBUF):
                ch = o * NBUF + b
                pltpu.make_async_copy(
                    y_hbm.at[si_v.at[ch]], rows_v.at[b], sems.at[b]
                ).wait()
                pltpu.sync_copy(rows_v.at[b], acc_sh.at[di_v.at[ch]], add=True)
                nch = ch + NBUF

                @pl.when(nch < CHS)
                def _():
                    pltpu.async_copy(
                        y_hbm.at[si_v.at[nch]], rows_v.at[b], sems.at[b]
                    )
            return 0

        lax.fori_loop(0, CHS // NBUF, outer, 0)
    plsc.subcore_barrier()

    for off, sz in ((0, C), (C, C), (2 * C, C), (3 * C, C), (4 * C, RPT - 4 * C)):
        pltpu.sync_copy(
            acc_sh.at[pl.ds(sid * RPT + off, sz)],
            rows_v.at[0, pl.ds(0, sz)],
        )
        pltpu.sync_copy(
            rows_v.at[0, pl.ds(0, sz)],
            out_hbm.at[pl.ds(cid * AR + sid * RPT + off, sz)],
        )


def _tc_xw(x, W1):
    def body(x_ref, w_ref, xw_ref):
        xw_ref[...] = jnp.dot(x_ref[...], w_ref[...],
                              preferred_element_type=jnp.float32,
                              precision=lax.Precision.HIGHEST)

    return pl.pallas_call(
        body,
        grid=(NG,),
        in_specs=[
            pl.BlockSpec((BR, D), lambda i: (i, 0)),
            pl.BlockSpec((D, D), lambda i: (0, 0)),
        ],
        out_specs=pl.BlockSpec((BR, D), lambda i: (i, 0)),
        out_shape=jax.ShapeDtypeStruct((N, D), jnp.float32),
    )(x, W1)


def _tc_scale(deg_col, xw):
    def body(deg_ref, xw_ref, y_ref, dis_ref):
        dis = lax.rsqrt(deg_ref[...] + 1.0)
        y_ref[...] = xw_ref[...] * dis
        dis_ref[...] = dis

    return pl.pallas_call(
        body,
        grid=(NG,),
        in_specs=[
            pl.BlockSpec((BR, 1), lambda i: (i, 0)),
            pl.BlockSpec((BR, D), lambda i: (i, 0)),
        ],
        out_specs=[
            pl.BlockSpec((BR, D), lambda i: (i, 0)),
            pl.BlockSpec((BR, 1), lambda i: (i, 0)),
        ],
        out_shape=[
            jax.ShapeDtypeStruct((N, D), jnp.float32),
            jax.ShapeDtypeStruct((N, 1), jnp.float32),
        ],
    )(deg_col, xw)


def _tc_mid(acc, y, dis, b, w):
    def body(a_ref, y_ref, d_ref, b_ref, w_ref, h_ref, y2_ref):
        d = d_ref[...]
        h = jnp.maximum((a_ref[...] + y_ref[...]) * d + b_ref[...], 0.0)
        h_ref[...] = h
        y2_ref[...] = jnp.dot(h, w_ref[...], preferred_element_type=jnp.float32,
                              precision=lax.Precision.HIGHEST) * d

    return pl.pallas_call(
        body,
        grid=(NG,),
        in_specs=[
            pl.BlockSpec((BR, D), lambda i: (i, 0)),
            pl.BlockSpec((BR, D), lambda i: (i, 0)),
            pl.BlockSpec((BR, 1), lambda i: (i, 0)),
            pl.BlockSpec((D,), lambda i: (0,)),
            pl.BlockSpec((D, D), lambda i: (0, 0)),
        ],
        out_specs=[
            pl.BlockSpec((BR, D), lambda i: (i, 0)),
            pl.BlockSpec((BR, D), lambda i: (i, 0)),
        ],
        out_shape=[
            jax.ShapeDtypeStruct((N, D), jnp.float32),
            jax.ShapeDtypeStruct((N, D), jnp.float32),
        ],
    )(acc, y, dis, b, w)


def _tc_post(h2, batch_col, lin_wT, lin_b2):
    def body(h_ref, bat, lwT, lb, sums_ref, counts_ref, out_ref):
        i = pl.program_id(0)
        h = h_ref[...]
        g_iota = lax.broadcasted_iota(jnp.int32, (1, G), 1)
        oh = (bat[...] == g_iota).astype(jnp.float32)
        psum = lax.dot_general(
            oh, h, (((0,), (0,)), ((), ())),
            preferred_element_type=jnp.float32,
            precision=lax.Precision.HIGHEST,
        )
        pcnt = jnp.sum(oh, axis=0)[:, None]

        @pl.when(i == 0)
        def _():
            sums_ref[...] = psum
            counts_ref[...] = pcnt

        @pl.when(i > 0)
        def _():
            sums_ref[...] += psum
            counts_ref[...] += pcnt

        @pl.when(i == NG - 1)
        def _():
            pooled = sums_ref[...] / jnp.maximum(counts_ref[...], 1.0)
            out_ref[...] = (
                jnp.sum(pooled * lwT[...], axis=1, keepdims=True) + lb[...]
            )

    _, _, out = pl.pallas_call(
        body,
        grid=(NG,),
        in_specs=[
            pl.BlockSpec((BR, D), lambda i: (i, 0)),
            pl.BlockSpec((BR, 1), lambda i: (i, 0)),
            pl.BlockSpec((1, D), lambda i: (0, 0)),
            pl.BlockSpec((1, 1), lambda i: (0, 0)),
        ],
        out_specs=[
            pl.BlockSpec((G, D), lambda i: (0, 0)),
            pl.BlockSpec((G, 1), lambda i: (0, 0)),
            pl.BlockSpec((G, 1), lambda i: (0, 0)),
        ],
        out_shape=[
            jax.ShapeDtypeStruct((G, D), jnp.float32),
            jax.ShapeDtypeStruct((G, 1), jnp.float32),
            jax.ShapeDtypeStruct((G, 1), jnp.float32),
        ],
    )(h2, batch_col, lin_wT, lin_b2)
    return out


def kernel(x, edge_index, batch, W1, b1, W2, b2, lin_w, lin_b):
    src = edge_index[0]
    dst = edge_index[1]
    src_r = src.reshape(NS * NST, CHS, C)
    dst_r = dst.reshape(NS, NCH, C)

    dst0 = jnp.where(dst < HN, dst, HN + (dst & (TR - 1)))
    dst1 = jnp.where(dst >= HN, dst - HN + TR, dst & (TR - 1))
    dstT = jnp.concatenate(
        [dst0.reshape(NS * NST, CHS, C), dst1.reshape(NS * NST, CHS, C)], axis=0
    )

    xw = _tc_xw(x, W1)
    deg = _deg_kernel(dst_r)
    y1, dis = _tc_scale(deg[:N, None], xw)

    zeros = jnp.zeros((RPT, D), jnp.float32)
    bs = jnp.stack([b1, b2])

    def cond(carry):
        return carry[0] < 2

    def body(carry):
        i, y, _ = carry
        accs = _edge_kernel(y, zeros, src_r, dstT)
        acc = jnp.concatenate(
            [accs[:HN], accs[AR + TR:AR + TR + (N - HN)]], axis=0
        )
        b = lax.dynamic_index_in_dim(bs, i, 0, keepdims=False)
        h, y_next = _tc_mid(acc, y, dis, b, W2)
        return (i + 1, y_next, h)

    _, _, h2 = lax.while_loop(
        cond, body, (0, y1, jnp.zeros((N, D), jnp.float32))
    )

    return _tc_post(h2, batch[:, None], lin_w.T, lin_b[:, None])
